# baseline (speedup 1.0000x reference)
import dataclasses
import functools

import jax
import jax.numpy as jnp
from jax import lax
from jax.experimental import pallas as pl
from jax.experimental.pallas import tpu as pltpu
from jax.experimental.pallas import tpu_sc as plsc

NC = 2
NS = 16
NW = NC * NS
L = 16

B = 16384
D = 128
CHUNK = 128
B_PER_W = B // NW
N_CHUNKS = B_PER_W // CHUNK
QROWS = 4


def _gmf_sc(user_ids, item_ids, user_table, item_table, W, b_vec):
    mesh = plsc.VectorSubcoreMesh(core_axis_name="c", subcore_axis_name="s")

    cp = pltpu.CompilerParams()
    if "needs_layout_passes" in pltpu.CompilerParams.__dataclass_fields__:
        cp = dataclasses.replace(cp, needs_layout_passes=False)

    @functools.partial(
        pl.kernel,
        compiler_params=cp,
        out_type=jax.ShapeDtypeStruct((B,), jnp.float32),
        mesh=mesh,
        scratch_types=[
            pltpu.VMEM((B_PER_W,), jnp.int32),
            pltpu.VMEM((B_PER_W,), jnp.int32),
            pltpu.VMEM((CHUNK, D), jnp.float32),
            pltpu.VMEM((CHUNK, D), jnp.float32),
            pltpu.VMEM((CHUNK, D), jnp.float32),
            pltpu.VMEM((CHUNK, D), jnp.float32),
            pltpu.VMEM((B_PER_W,), jnp.float32),
            pltpu.VMEM((D,), jnp.float32),
            pltpu.VMEM((L,), jnp.float32),
            pltpu.SemaphoreType.DMA,
            pltpu.SemaphoreType.DMA,
            pltpu.SemaphoreType.DMA,
            pltpu.SemaphoreType.DMA,
            pltpu.SemaphoreType.DMA,
        ],
    )
    def k(uids_hbm, iids_hbm, utab_hbm, itab_hbm, w_hbm, b_hbm, out_hbm,
          uidx_v, iidx_v, u0, u1, i0, i1, out_v, w_v, b_v,
          su0, su1, si0, si1, sp):
        wid = lax.axis_index("s") * NC + lax.axis_index("c")
        base = wid * B_PER_W

        cp_ui = pltpu.async_copy(uids_hbm.at[pl.ds(base, B_PER_W)], uidx_v, sp)
        cp_ii = pltpu.async_copy(iids_hbm.at[pl.ds(base, B_PER_W)], iidx_v, sp)
        cp_ui.wait()
        cp_ii.wait()

        u_bufs, i_bufs = [u0, u1], [i0, i1]
        u_sems, i_sems = [su0, su1], [si0, si1]

        def start(c):
            s = c % 2
            cu = pltpu.async_copy(
                utab_hbm.at[uidx_v.at[pl.ds(c * CHUNK, CHUNK)]],
                u_bufs[s], u_sems[s])
            ci = pltpu.async_copy(
                itab_hbm.at[iidx_v.at[pl.ds(c * CHUNK, CHUNK)]],
                i_bufs[s], i_sems[s])
            return cu, ci

        cps = [start(0)]

        cp_w = pltpu.async_copy(w_hbm.at[0], w_v, sp)
        cp_b = pltpu.async_copy(b_hbm, b_v, sp)
        cp_w.wait()
        cp_b.wait()
        w_regs = [w_v[pl.ds(L * j, L)] for j in range(D // L)]
        bv = b_v[...]

        for c in range(N_CHUNKS):
            s = c % 2
            cu, ci = cps[c]
            if c + 1 < N_CHUNKS:
                cps.append(start(c + 1))
            cu.wait()
            ci.wait()
            urows_v, irows_v = u_bufs[s], i_bufs[s]

            @pl.loop(0, CHUNK // L)
            def _group(g, c=c, urows_v=urows_v, irows_v=irows_v):
                lane = lax.iota(jnp.int32, L)

                def quad(qi, out_vec):
                    rbase = g * L + qi * QROWS
                    for r in range(QROWS):
                        acc = jnp.zeros((L,), jnp.float32)
                        for j in range(D // L):
                            u = urows_v[rbase + r, pl.ds(L * j, L)]
                            v = irows_v[rbase + r, pl.ds(L * j, L)]
                            acc = acc + (u * v) * w_regs[j]
                        su = jnp.sum(acc)
                        out_vec = jnp.where(lane == qi * QROWS + r, su,
                                            out_vec)
                    return out_vec

                out_vec = lax.fori_loop(0, L // QROWS, quad,
                                        jnp.zeros((L,), jnp.float32))
                x = out_vec + bv
                y = 1.0 / (1.0 + jnp.exp(-x))
                out_v[pl.ds(c * CHUNK + g * L, L)] = y

        pltpu.sync_copy(out_v, out_hbm.at[pl.ds(base, B_PER_W)])

    return k(user_ids, item_ids, user_table, item_table, W, b_vec)


def kernel(user_ids, item_ids, user_table, item_table, W, b):
    b_vec = jnp.broadcast_to(b.astype(jnp.float32), (L,))
    out = _gmf_sc(user_ids, item_ids, user_table, item_table, W, b_vec)
    return out.reshape(B, 1)
